# Initial kernel scaffold; baseline (speedup 1.0000x reference)
import jax
import jax.numpy as jnp
from jax.experimental import pallas as pl


def kernel(S, A, block_id, block_table, atom_table):
    raise NotImplementedError("write your pallas kernel here")


# speedup vs baseline: 8.0710x; 8.0710x over previous
import functools

import jax
import jax.numpy as jnp
from jax import lax
from jax.experimental import pallas as pl
from jax.experimental.pallas import tpu as pltpu
from jax.experimental.pallas import tpu_sc as plsc

NB = 50000
NU = 400000
NUM_BLOCK_TYPE = 32
NUM_ATOM_TYPE = 128
EMBED = 128

NC = 2
NS = 16
NW = NC * NS
L = 16

CHUNK = 128
NCHUNK = NU // CHUNK
BASE_CHUNKS = NCHUNK // NW
EXTRA = NCHUNK % NW


def _build_fused(block_table, atom_table):
    def body(b_ref, a_ref, o_ref):
        t = pl.program_id(0)
        o_ref[...] = a_ref[...] + b_ref[pl.ds(t, 1), :]

    return pl.pallas_call(
        body,
        grid=(NUM_BLOCK_TYPE,),
        in_specs=[
            pl.BlockSpec((NUM_BLOCK_TYPE, EMBED), lambda i: (0, 0)),
            pl.BlockSpec((NUM_ATOM_TYPE, EMBED), lambda i: (0, 0)),
        ],
        out_specs=pl.BlockSpec((NUM_ATOM_TYPE, EMBED), lambda i: (i, 0)),
        out_shape=jax.ShapeDtypeStruct(
            (NUM_BLOCK_TYPE * NUM_ATOM_TYPE, EMBED), jnp.float32
        ),
    )(block_table, atom_table)


def _make_sc_gather():
    mesh = plsc.VectorSubcoreMesh(core_axis_name="c", subcore_axis_name="s")

    @functools.partial(
        pl.kernel,
        mesh=mesh,
        out_type=jax.ShapeDtypeStruct((NU, EMBED), jnp.float32),
        scratch_types=[
            pltpu.VMEM((CHUNK,), jnp.int32),
            pltpu.VMEM((CHUNK,), jnp.int32),
            pltpu.VMEM((CHUNK,), jnp.int32),
            pltpu.VMEM((CHUNK,), jnp.int32),
            pltpu.VMEM((CHUNK, EMBED), jnp.float32),
            pltpu.SemaphoreType.DMA,
            pltpu.SemaphoreType.DMA,
        ],
    )
    def sc_gather(
        s_hbm, bid_hbm, a_hbm, fused_hbm, out_hbm,
        bid_v, a_v, t_v, f_v, rows_v, sem_t, sem_r,
    ):
        cid = lax.axis_index("c")
        sid = lax.axis_index("s")
        wid = sid * NC + cid
        n_chunks = BASE_CHUNKS + jnp.where(wid < EXTRA, 1, 0)
        start = wid * BASE_CHUNKS + jnp.minimum(wid, EXTRA)

        @pl.loop(0, n_chunks)
        def _chunk(i):
            row0 = (start + i) * CHUNK
            pltpu.sync_copy(bid_hbm.at[pl.ds(row0, CHUNK)], bid_v)
            pltpu.sync_copy(a_hbm.at[pl.ds(row0, CHUNK)], a_v)
            pltpu.async_copy(s_hbm.at[bid_v], t_v, sem_t).wait()
            for g in range(CHUNK // L):
                t16 = t_v[pl.ds(g * L, L)]
                a16 = a_v[pl.ds(g * L, L)]
                f_v[pl.ds(g * L, L)] = t16 * EMBED + a16
            pltpu.async_copy(fused_hbm.at[f_v], rows_v, sem_r).wait()
            pltpu.sync_copy(rows_v, out_hbm.at[pl.ds(row0, CHUNK)])

    return sc_gather


_sc_gather = _make_sc_gather()


@jax.jit
def kernel(S, A, block_id, block_table, atom_table):
    fused = _build_fused(block_table, atom_table)
    return _sc_gather(S, block_id, A, fused)


# speedup vs baseline: 10.4547x; 1.2953x over previous
import functools

import jax
import jax.numpy as jnp
from jax import lax
from jax.experimental import pallas as pl
from jax.experimental.pallas import tpu as pltpu
from jax.experimental.pallas import tpu_sc as plsc

NB = 50000
NU = 400000
NUM_BLOCK_TYPE = 32
NUM_ATOM_TYPE = 128
EMBED = 128

NC = 2
NS = 16
NW = NC * NS
L = 16

CHUNK = 128
NCHUNK = NU // CHUNK
BASE_CHUNKS = NCHUNK // NW
EXTRA = NCHUNK % NW


def _build_fused(block_table, atom_table):
    def body(b_ref, a_ref, o_ref):
        t = pl.program_id(0)
        o_ref[...] = a_ref[...] + b_ref[pl.ds(t, 1), :]

    return pl.pallas_call(
        body,
        grid=(NUM_BLOCK_TYPE,),
        in_specs=[
            pl.BlockSpec((NUM_BLOCK_TYPE, EMBED), lambda i: (0, 0)),
            pl.BlockSpec((NUM_ATOM_TYPE, EMBED), lambda i: (0, 0)),
        ],
        out_specs=pl.BlockSpec((NUM_ATOM_TYPE, EMBED), lambda i: (i, 0)),
        out_shape=jax.ShapeDtypeStruct(
            (NUM_BLOCK_TYPE * NUM_ATOM_TYPE, EMBED), jnp.float32
        ),
    )(block_table, atom_table)


def _make_sc_gather():
    mesh = plsc.VectorSubcoreMesh(core_axis_name="c", subcore_axis_name="s")
    R = (NCHUNK + NW - 1) // NW

    @functools.partial(
        pl.kernel,
        mesh=mesh,
        out_type=jax.ShapeDtypeStruct((NU, EMBED), jnp.float32),
        scratch_types=[
            pltpu.VMEM((CHUNK,), jnp.int32),
            pltpu.VMEM((CHUNK,), jnp.int32),
            pltpu.VMEM((CHUNK,), jnp.int32),
            pltpu.VMEM((CHUNK,), jnp.int32),
            pltpu.VMEM((CHUNK,), jnp.int32),
            pltpu.VMEM((CHUNK,), jnp.int32),
            pltpu.VMEM((CHUNK, EMBED), jnp.float32),
            pltpu.VMEM((CHUNK, EMBED), jnp.float32),
            pltpu.SemaphoreType.DMA,
            pltpu.SemaphoreType.DMA,
            pltpu.SemaphoreType.DMA,
            pltpu.SemaphoreType.DMA,
            pltpu.SemaphoreType.DMA,
        ],
    )
    def sc_gather(
        s_hbm, bid_hbm, a_hbm, fused_hbm, out_hbm,
        bid0, bid1, a0, a1, t_v, f_v, rows0, rows1,
        sin0, sin1, sem_g, so0, so1,
    ):
        bid_vs = (bid0, bid1)
        a_vs = (a0, a1)
        rows_vs = (rows0, rows1)
        sem_in = (sin0, sin1)
        sem_o = (so0, so1)

        cid = lax.axis_index("c")
        sid = lax.axis_index("s")
        wid = sid * NC + cid

        def chunk_of(r):
            return r * NW + wid

        def active(r):
            return jnp.logical_and(r >= 0, chunk_of(r) < NCHUNK)

        def row0_of(r):
            return chunk_of(r) * CHUNK

        def fire_in(r, b):
            @pl.when(active(r))
            def _():
                row0 = row0_of(r)
                pltpu.async_copy(bid_hbm.at[pl.ds(row0, CHUNK)], bid_vs[b], sem_in[b])
                pltpu.async_copy(a_hbm.at[pl.ds(row0, CHUNK)], a_vs[b], sem_in[b])

        def do_round(r, b):
            @pl.when(active(r))
            def _():
                row0 = row0_of(r)
                pltpu.make_async_copy(
                    bid_hbm.at[pl.ds(row0, CHUNK)], bid_vs[b], sem_in[b]
                ).wait()
                pltpu.make_async_copy(
                    a_hbm.at[pl.ds(row0, CHUNK)], a_vs[b], sem_in[b]
                ).wait()
                pltpu.async_copy(s_hbm.at[bid_vs[b]], t_v, sem_g).wait()
                for g in range(CHUNK // L):
                    t16 = t_v[pl.ds(g * L, L)]
                    a16 = a_vs[b][pl.ds(g * L, L)]
                    f_v[pl.ds(g * L, L)] = t16 * EMBED + a16

            fire_in(r + 2, b)

            @pl.when(active(r - 2))
            def _():
                pltpu.make_async_copy(
                    rows_vs[b], out_hbm.at[pl.ds(row0_of(r - 2), CHUNK)], sem_o[b]
                ).wait()

            @pl.when(active(r))
            def _():
                pltpu.async_copy(fused_hbm.at[f_v], rows_vs[b], sem_g).wait()
                pltpu.async_copy(
                    rows_vs[b], out_hbm.at[pl.ds(row0_of(r), CHUNK)], sem_o[b]
                )

        fire_in(0, 0)
        fire_in(1, 1)

        @pl.loop(0, R, step=2)
        def _body(r0):
            for db in range(2):
                do_round(r0 + db, db)

        for b, r in ((0, R - 2), (1, R - 1)):
            @pl.when(active(r))
            def _drain():
                pltpu.make_async_copy(
                    rows_vs[b], out_hbm.at[pl.ds(row0_of(r), CHUNK)], sem_o[b]
                ).wait()

    return sc_gather


_sc_gather = _make_sc_gather()


@jax.jit
def kernel(S, A, block_id, block_table, atom_table):
    fused = _build_fused(block_table, atom_table)
    return _sc_gather(S, block_id, A, fused)


# speedup vs baseline: 11.5783x; 1.1075x over previous
import functools

import jax
import jax.numpy as jnp
from jax import lax
from jax.experimental import pallas as pl
from jax.experimental.pallas import tpu as pltpu
from jax.experimental.pallas import tpu_sc as plsc

NB = 50000
NU = 400000
NUM_BLOCK_TYPE = 32
NUM_ATOM_TYPE = 128
EMBED = 128

NC = 2
NS = 16
NW = NC * NS
L = 16

CHUNK = 128
NCHUNK = NU // CHUNK
BASE_CHUNKS = NCHUNK // NW
EXTRA = NCHUNK % NW


def _build_fused(block_table, atom_table):
    def body(b_ref, a_ref, o_ref):
        t = pl.program_id(0)
        o_ref[...] = a_ref[...] + b_ref[pl.ds(t, 1), :]

    return pl.pallas_call(
        body,
        grid=(NUM_BLOCK_TYPE,),
        in_specs=[
            pl.BlockSpec((NUM_BLOCK_TYPE, EMBED), lambda i: (0, 0)),
            pl.BlockSpec((NUM_ATOM_TYPE, EMBED), lambda i: (0, 0)),
        ],
        out_specs=pl.BlockSpec((NUM_ATOM_TYPE, EMBED), lambda i: (i, 0)),
        out_shape=jax.ShapeDtypeStruct(
            (NUM_BLOCK_TYPE * NUM_ATOM_TYPE, EMBED), jnp.float32
        ),
    )(block_table, atom_table)


K = 4
SSTEP = NW * K
NSUPER = (NCHUNK + SSTEP - 1) // SSTEP


def _make_sc_gather():
    mesh = plsc.VectorSubcoreMesh(core_axis_name="c", subcore_axis_name="s")
    NSUPER2 = NSUPER + (NSUPER % 2)

    @functools.partial(
        pl.kernel,
        mesh=mesh,
        out_type=jax.ShapeDtypeStruct((NU, EMBED), jnp.float32),
        scratch_types=[
            pltpu.VMEM((K, CHUNK), jnp.int32),
            pltpu.VMEM((K, CHUNK), jnp.int32),
            pltpu.VMEM((K, CHUNK), jnp.int32),
            pltpu.VMEM((K, CHUNK), jnp.int32),
            pltpu.VMEM((K, CHUNK), jnp.int32),
            pltpu.VMEM((K, CHUNK), jnp.int32),
            pltpu.VMEM((K * CHUNK, EMBED), jnp.float32),
            pltpu.SemaphoreType.DMA,
            pltpu.SemaphoreType.DMA,
            pltpu.SemaphoreType.DMA,
            (pltpu.SemaphoreType.DMA,) * K,
            (pltpu.SemaphoreType.DMA,) * K,
        ],
    )
    def sc_gather(
        s_hbm, bid_hbm, a_hbm, fused_hbm, out_hbm,
        bid0, bid1, a0, a1, t_v, f_v, rows_v,
        sin0, sin1, sem_t, sem_g, sem_o,
    ):
        bid_vs = (bid0, bid1)
        a_vs = (a0, a1)
        sem_in = (sin0, sin1)

        cid = lax.axis_index("c")
        sid = lax.axis_index("s")
        wid = sid * NC + cid

        def chunk_of(s, k):
            return s * SSTEP + wid * K + k

        def active(s, k):
            return jnp.logical_and(s >= 0, chunk_of(s, k) < NCHUNK)

        def row0_of(s, k):
            return chunk_of(s, k) * CHUNK

        def fire_in(s, b):
            for k in range(K):
                @pl.when(active(s, k))
                def _():
                    row0 = row0_of(s, k)
                    pltpu.async_copy(
                        bid_hbm.at[pl.ds(row0, CHUNK)], bid_vs[b].at[k], sem_in[b]
                    )
                    pltpu.async_copy(
                        a_hbm.at[pl.ds(row0, CHUNK)], a_vs[b].at[k], sem_in[b]
                    )

        def do_superstep(s, b):
            for k in range(K):
                @pl.when(active(s, k))
                def _():
                    row0 = row0_of(s, k)
                    pltpu.make_async_copy(
                        bid_hbm.at[pl.ds(row0, CHUNK)], bid_vs[b].at[k], sem_in[b]
                    ).wait()
                    pltpu.make_async_copy(
                        a_hbm.at[pl.ds(row0, CHUNK)], a_vs[b].at[k], sem_in[b]
                    ).wait()
                    pltpu.async_copy(s_hbm.at[bid_vs[b].at[k]], t_v.at[k], sem_t)

            for k in range(K):
                @pl.when(active(s, k))
                def _():
                    pltpu.make_async_copy(
                        s_hbm.at[bid_vs[b].at[k]], t_v.at[k], sem_t
                    ).wait()
                    for g in range(CHUNK // L):
                        t16 = t_v[k, pl.ds(g * L, L)]
                        a16 = a_vs[b][k, pl.ds(g * L, L)]
                        f_v[k, pl.ds(g * L, L)] = t16 * EMBED + a16

            fire_in(s + 2, b)

            for k in range(K):
                @pl.when(active(s - 1, k))
                def _():
                    pltpu.make_async_copy(
                        rows_v.at[pl.ds(k * CHUNK, CHUNK)],
                        out_hbm.at[pl.ds(row0_of(s - 1, k), CHUNK)],
                        sem_o[k],
                    ).wait()

                @pl.when(active(s, k))
                def _():
                    pltpu.async_copy(
                        fused_hbm.at[f_v.at[k]],
                        rows_v.at[pl.ds(k * CHUNK, CHUNK)],
                        sem_g[k],
                    )

            for k in range(K):
                @pl.when(active(s, k))
                def _():
                    pltpu.make_async_copy(
                        fused_hbm.at[f_v.at[k]],
                        rows_v.at[pl.ds(k * CHUNK, CHUNK)],
                        sem_g[k],
                    ).wait()
                    pltpu.async_copy(
                        rows_v.at[pl.ds(k * CHUNK, CHUNK)],
                        out_hbm.at[pl.ds(row0_of(s, k), CHUNK)],
                        sem_o[k],
                    )

        fire_in(0, 0)
        fire_in(1, 1)

        @pl.loop(0, NSUPER2, step=2)
        def _body(s0):
            for db in range(2):
                do_superstep(s0 + db, db)

        for k in range(K):
            @pl.when(active(NSUPER2 - 1, k))
            def _drain():
                pltpu.make_async_copy(
                    rows_v.at[pl.ds(k * CHUNK, CHUNK)],
                    out_hbm.at[pl.ds(row0_of(NSUPER2 - 1, k), CHUNK)],
                    sem_o[k],
                ).wait()

    return sc_gather


_sc_gather = _make_sc_gather()


@jax.jit
def kernel(S, A, block_id, block_table, atom_table):
    fused = _build_fused(block_table, atom_table)
    return _sc_gather(S, block_id, A, fused)


# speedup vs baseline: 30.8202x; 2.6619x over previous
import functools

import jax
import jax.numpy as jnp
from jax import lax
from jax.experimental import pallas as pl
from jax.experimental.pallas import tpu as pltpu
from jax.experimental.pallas import tpu_sc as plsc

NB = 50000
NU = 400000
NUM_BLOCK_TYPE = 32
NUM_ATOM_TYPE = 128
EMBED = 128

NC = 2
NS = 16
NW = NC * NS
L = 16

CHUNK = 128
NCHUNK = NU // CHUNK
BASE_CHUNKS = NCHUNK // NW
EXTRA = NCHUNK % NW


def _build_fused(block_table, atom_table):
    def body(b_ref, a_ref, o_ref):
        t = pl.program_id(0)
        o_ref[...] = a_ref[...] + b_ref[pl.ds(t, 1), :]

    return pl.pallas_call(
        body,
        grid=(NUM_BLOCK_TYPE,),
        in_specs=[
            pl.BlockSpec((NUM_BLOCK_TYPE, EMBED), lambda i: (0, 0)),
            pl.BlockSpec((NUM_ATOM_TYPE, EMBED), lambda i: (0, 0)),
        ],
        out_specs=pl.BlockSpec((NUM_ATOM_TYPE, EMBED), lambda i: (i, 0)),
        out_shape=jax.ShapeDtypeStruct(
            (NUM_BLOCK_TYPE * NUM_ATOM_TYPE, EMBED), jnp.float32
        ),
    )(block_table, atom_table)


K = 4
SSTEP = NW * K
NSUPER = (NCHUNK + SSTEP - 1) // SSTEP


def _make_sc_gather():
    mesh = plsc.VectorSubcoreMesh(core_axis_name="c", subcore_axis_name="s")
    NSUPER2 = NSUPER + (NSUPER % 2)

    @functools.partial(
        pl.kernel,
        mesh=mesh,
        out_type=jax.ShapeDtypeStruct((NU, EMBED), jnp.float32),
        scratch_types=[
            pltpu.VMEM((K, CHUNK), jnp.int32),
            pltpu.VMEM((K, CHUNK), jnp.int32),
            pltpu.VMEM((K, CHUNK), jnp.int32),
            pltpu.VMEM((K, CHUNK), jnp.int32),
            pltpu.VMEM((K, CHUNK), jnp.int32),
            pltpu.VMEM((K, CHUNK), jnp.int32),
            pltpu.VMEM((K * CHUNK, EMBED), jnp.float32),
            pltpu.VMEM_SHARED(
                (NUM_BLOCK_TYPE * NUM_ATOM_TYPE, EMBED), jnp.float32
            ),
            pltpu.VMEM_SHARED((NB,), jnp.int32),
            pltpu.SemaphoreType.DMA,
            pltpu.SemaphoreType.DMA,
            pltpu.SemaphoreType.DMA,
            (pltpu.SemaphoreType.DMA,) * K,
            (pltpu.SemaphoreType.DMA,) * K,
        ],
    )
    def sc_gather(
        s_hbm, bid_hbm, a_hbm, fused_hbm, out_hbm,
        bid0, bid1, a0, a1, t_v, f_v, rows_v, fused_sh, s_sh,
        sin0, sin1, sem_t, sem_g, sem_o,
    ):
        bid_vs = (bid0, bid1)
        a_vs = (a0, a1)
        sem_in = (sin0, sin1)

        cid = lax.axis_index("c")
        sid = lax.axis_index("s")
        wid = sid * NC + cid

        def chunk_of(s, k):
            return s * SSTEP + wid * K + k

        def active(s, k):
            return jnp.logical_and(s >= 0, chunk_of(s, k) < NCHUNK)

        def row0_of(s, k):
            return chunk_of(s, k) * CHUNK

        def fire_in(s, b):
            for k in range(K):
                @pl.when(active(s, k))
                def _():
                    row0 = row0_of(s, k)
                    pltpu.async_copy(
                        bid_hbm.at[pl.ds(row0, CHUNK)], bid_vs[b].at[k], sem_in[b]
                    )
                    pltpu.async_copy(
                        a_hbm.at[pl.ds(row0, CHUNK)], a_vs[b].at[k], sem_in[b]
                    )

        def do_superstep(s, b):
            for k in range(K):
                @pl.when(active(s, k))
                def _():
                    row0 = row0_of(s, k)
                    pltpu.make_async_copy(
                        bid_hbm.at[pl.ds(row0, CHUNK)], bid_vs[b].at[k], sem_in[b]
                    ).wait()
                    pltpu.make_async_copy(
                        a_hbm.at[pl.ds(row0, CHUNK)], a_vs[b].at[k], sem_in[b]
                    ).wait()
                    pltpu.async_copy(s_sh.at[bid_vs[b].at[k]], t_v.at[k], sem_t)

            for k in range(K):
                @pl.when(active(s, k))
                def _():
                    pltpu.make_async_copy(
                        s_sh.at[bid_vs[b].at[k]], t_v.at[k], sem_t
                    ).wait()
                    for g in range(CHUNK // L):
                        t16 = t_v[k, pl.ds(g * L, L)]
                        a16 = a_vs[b][k, pl.ds(g * L, L)]
                        f_v[k, pl.ds(g * L, L)] = t16 * EMBED + a16

            fire_in(s + 2, b)

            for k in range(K):
                @pl.when(active(s - 1, k))
                def _():
                    pltpu.make_async_copy(
                        rows_v.at[pl.ds(k * CHUNK, CHUNK)],
                        out_hbm.at[pl.ds(row0_of(s - 1, k), CHUNK)],
                        sem_o[k],
                    ).wait()

                @pl.when(active(s, k))
                def _():
                    pltpu.async_copy(
                        fused_sh.at[f_v.at[k]],
                        rows_v.at[pl.ds(k * CHUNK, CHUNK)],
                        sem_g[k],
                    )

            for k in range(K):
                @pl.when(active(s, k))
                def _():
                    pltpu.make_async_copy(
                        fused_sh.at[f_v.at[k]],
                        rows_v.at[pl.ds(k * CHUNK, CHUNK)],
                        sem_g[k],
                    ).wait()
                    pltpu.async_copy(
                        rows_v.at[pl.ds(k * CHUNK, CHUNK)],
                        out_hbm.at[pl.ds(row0_of(s, k), CHUNK)],
                        sem_o[k],
                    )

        @pl.when(sid == 0)
        def _stage():
            pltpu.sync_copy(fused_hbm, fused_sh)
            pltpu.sync_copy(s_hbm, s_sh)

        plsc.subcore_barrier()

        fire_in(0, 0)
        fire_in(1, 1)

        @pl.loop(0, NSUPER2, step=2)
        def _body(s0):
            for db in range(2):
                do_superstep(s0 + db, db)

        for k in range(K):
            @pl.when(active(NSUPER2 - 1, k))
            def _drain():
                pltpu.make_async_copy(
                    rows_v.at[pl.ds(k * CHUNK, CHUNK)],
                    out_hbm.at[pl.ds(row0_of(NSUPER2 - 1, k), CHUNK)],
                    sem_o[k],
                ).wait()

    return sc_gather


_sc_gather = _make_sc_gather()


@jax.jit
def kernel(S, A, block_id, block_table, atom_table):
    fused = _build_fused(block_table, atom_table)
    return _sc_gather(S, block_id, A, fused)
